# baseline (speedup 1.0000x reference)
import functools

import jax
import jax.numpy as jnp
from jax import lax
from jax.experimental import pallas as pl
from jax.experimental.pallas import tpu as pltpu
from jax.experimental.pallas import tpu_sc as plsc

D = 32
NC = 2
NS = 16
NW = NC * NS
G = 16


def _sc_neg(ctx_emb, vj, neg_flat, K):
    B = vj.shape[0]
    b_per_w = B // NW
    n_per_w = b_per_w * K
    CH = G * K
    n_chunks = b_per_w // G
    mesh = plsc.VectorSubcoreMesh(core_axis_name="c", subcore_axis_name="s")

    @functools.partial(
        pl.kernel,
        mesh=mesh,
        out_type=jax.ShapeDtypeStruct((K, B), jnp.float32),
        scratch_types=[
            pltpu.VMEM((b_per_w,), jnp.int32),
            pltpu.VMEM((n_per_w,), jnp.int32),
            pltpu.VMEM((G, D), jnp.float32),
            pltpu.VMEM((G, D), jnp.float32),
            pltpu.VMEM((CH, D), jnp.float32),
            pltpu.VMEM((CH, D), jnp.float32),
            pltpu.VMEM((D, G), jnp.float32),
            pltpu.VMEM((K, b_per_w), jnp.float32),
            pltpu.SemaphoreType.DMA,
            pltpu.SemaphoreType.DMA,
            pltpu.SemaphoreType.DMA,
        ],
        compiler_params=pltpu.CompilerParams(
            disable_bounds_checks=True, needs_layout_passes=False),
    )
    def k(ctx_hbm, vj_hbm, neg_hbm, out_hbm,
          vj_idx, ng_idx, vjb0, vjb1, nb0, nb1, vjt, sc_v, sem, sn0, sn1):
        wid = lax.axis_index("s") * NC + lax.axis_index("c")
        base = wid * b_per_w
        lanes = lax.iota(jnp.int32, G)
        lanes_k = lanes * K

        pltpu.async_copy(vj_hbm.at[pl.ds(base, b_per_w)], vj_idx, sem)
        pltpu.async_copy(neg_hbm.at[pl.ds(base * K, n_per_w)], ng_idx, sem)
        pltpu.make_async_copy(vj_hbm.at[pl.ds(0, b_per_w)], vj_idx, sem).wait()
        pltpu.make_async_copy(neg_hbm.at[pl.ds(0, n_per_w)], ng_idx,
                              sem).wait()

        def fire_chunk(c, vjb, ngb, s):
            idxw = vj_idx[pl.ds(c * G, G)]
            for l in range(G):
                pltpu.async_copy(ctx_hbm.at[pl.ds(idxw[l], 1), :],
                                 vjb.at[pl.ds(l, 1), :], s)
            co = c * CH

            @pl.loop(0, CH, step=G)
            def _(rr):
                idxn = ng_idx[pl.ds(co + rr, G)]
                for l in range(G):
                    pltpu.async_copy(ctx_hbm.at[pl.ds(idxn[l], 1), :],
                                     ngb.at[pl.ds(rr + l, 1), :], s)

        def drain_chunk(ngb, s):
            @pl.loop(0, CH + G)
            def _(r):
                pltpu.make_async_copy(
                    ctx_hbm.at[pl.ds(0, 1), :],
                    ngb.at[pl.ds(0, 1), :], s).wait()

        def compute(c, vjb, ngb):
            g = c * G
            for d in range(D):
                cd = jnp.full((G,), d, jnp.int32)
                vjt[d, :] = plsc.load_gather(vjb, [lanes, cd])

            @pl.loop(0, K)
            def _(kk):
                rows_k = lanes_k + kk
                acc = jnp.zeros((G,), jnp.float32)
                for d in range(D):
                    cd = jnp.full((G,), d, jnp.int32)
                    nc = plsc.load_gather(ngb, [rows_k, cd])
                    acc = acc + nc * vjt[d, :]
                sc_v[kk, pl.ds(g, G)] = acc

        fire_chunk(0, vjb0, nb0, sn0)

        @pl.loop(0, n_chunks)
        def _(c):
            @pl.when(c % 2 == 0)
            def _():
                @pl.when(c + 1 < n_chunks)
                def _():
                    fire_chunk(c + 1, vjb1, nb1, sn1)
                drain_chunk(nb0, sn0)
                compute(c, vjb0, nb0)

            @pl.when(c % 2 == 1)
            def _():
                @pl.when(c + 1 < n_chunks)
                def _():
                    fire_chunk(c + 1, vjb0, nb0, sn0)
                drain_chunk(nb1, sn1)
                compute(c, vjb1, nb1)

        pltpu.sync_copy(sc_v, out_hbm.at[:, pl.ds(base, b_per_w)])

    return k(ctx_emb, vj, neg_flat)


def _sc_pos(node_emb, ctx_emb, vi, vj):
    B = vi.shape[0]
    b_per_w = B // NW
    n_chunks = b_per_w // G
    mesh = plsc.VectorSubcoreMesh(core_axis_name="c", subcore_axis_name="s")

    @functools.partial(
        pl.kernel,
        mesh=mesh,
        out_type=jax.ShapeDtypeStruct((1, B), jnp.float32),
        scratch_types=[
            pltpu.VMEM((b_per_w,), jnp.int32),
            pltpu.VMEM((b_per_w,), jnp.int32),
            pltpu.VMEM((G, D), jnp.float32),
            pltpu.VMEM((G, D), jnp.float32),
            pltpu.VMEM((G, D), jnp.float32),
            pltpu.VMEM((G, D), jnp.float32),
            pltpu.VMEM((1, b_per_w), jnp.float32),
            pltpu.SemaphoreType.DMA,
            pltpu.SemaphoreType.DMA,
            pltpu.SemaphoreType.DMA,
        ],
        compiler_params=pltpu.CompilerParams(
            disable_bounds_checks=True, needs_layout_passes=False),
    )
    def k(node_hbm, ctx_hbm, vi_hbm, vj_hbm, out_hbm,
          vi_idx, vj_idx, vib0, vib1, vjb0, vjb1, sc_v, sem, sn0, sn1):
        wid = lax.axis_index("s") * NC + lax.axis_index("c")
        base = wid * b_per_w
        lanes = lax.iota(jnp.int32, G)

        pltpu.async_copy(vi_hbm.at[pl.ds(base, b_per_w)], vi_idx, sem)
        pltpu.async_copy(vj_hbm.at[pl.ds(base, b_per_w)], vj_idx, sem)
        pltpu.make_async_copy(vi_hbm.at[pl.ds(0, b_per_w)], vi_idx, sem).wait()
        pltpu.make_async_copy(vj_hbm.at[pl.ds(0, b_per_w)], vj_idx, sem).wait()

        def fire_chunk(c, vib, vjb, s):
            idxv = vi_idx[pl.ds(c * G, G)]
            idxw = vj_idx[pl.ds(c * G, G)]
            for l in range(G):
                pltpu.async_copy(node_hbm.at[pl.ds(idxv[l], 1), :],
                                 vib.at[pl.ds(l, 1), :], s)
                pltpu.async_copy(ctx_hbm.at[pl.ds(idxw[l], 1), :],
                                 vjb.at[pl.ds(l, 1), :], s)

        def drain_chunk(vib, s):
            @pl.loop(0, 2 * G)
            def _(r):
                pltpu.make_async_copy(
                    node_hbm.at[pl.ds(0, 1), :],
                    vib.at[pl.ds(0, 1), :], s).wait()

        def compute(c, vib, vjb):
            g = c * G
            pos = jnp.zeros((G,), jnp.float32)
            for d in range(D):
                cd = jnp.full((G,), d, jnp.int32)
                vjc = plsc.load_gather(vjb, [lanes, cd])
                vic = plsc.load_gather(vib, [lanes, cd])
                pos = pos + vic * vjc
            sc_v[0, pl.ds(g, G)] = pos

        fire_chunk(0, vib0, vjb0, sn0)

        @pl.loop(0, n_chunks)
        def _(c):
            @pl.when(c % 2 == 0)
            def _():
                @pl.when(c + 1 < n_chunks)
                def _():
                    fire_chunk(c + 1, vib1, vjb1, sn1)
                drain_chunk(vib0, sn0)
                compute(c, vib0, vjb0)

            @pl.when(c % 2 == 1)
            def _():
                @pl.when(c + 1 < n_chunks)
                def _():
                    fire_chunk(c + 1, vib0, vjb0, sn0)
                drain_chunk(vib1, sn1)
                compute(c, vib1, vjb1)

        pltpu.sync_copy(sc_v, out_hbm.at[:, pl.ds(base, b_per_w)])

    return k(node_emb, ctx_emb, vi, vj)


def _tc_loss(negs, poss, B, K):
    BLK = 2048
    grid = B // BLK

    def body(n_ref, p_ref, out_ref):
        i = pl.program_id(0)
        part = (jnp.sum(jax.nn.log_sigmoid(-n_ref[...]))
                + jnp.sum(jax.nn.sigmoid(p_ref[...])))

        @pl.when(i == 0)
        def _():
            out_ref[0, 0] = 0.0

        out_ref[0, 0] += part

    out = pl.pallas_call(
        body,
        grid=(grid,),
        in_specs=[pl.BlockSpec((K, BLK), lambda i: (0, i)),
                  pl.BlockSpec((1, BLK), lambda i: (0, i))],
        out_specs=pl.BlockSpec(memory_space=pltpu.SMEM),
        out_shape=jax.ShapeDtypeStruct((1, 1), jnp.float32),
    )(negs, poss)
    return out[0, 0]


@jax.jit
def kernel(v_i, v_j, negative_samples, node_embeddings, context_embeddings):
    B, K = negative_samples.shape
    neg_flat = negative_samples.reshape(-1)
    negs = _sc_neg(context_embeddings, v_j, neg_flat, K)
    poss = _sc_pos(node_embeddings, context_embeddings, v_i, v_j)
    total = _tc_loss(negs, poss, B, K)
    return -(total / B)


# speedup vs baseline: 1.0047x; 1.0047x over previous
import functools

import jax
import jax.numpy as jnp
from jax import lax
from jax.experimental import pallas as pl
from jax.experimental.pallas import tpu as pltpu
from jax.experimental.pallas import tpu_sc as plsc

D = 32
NC = 2
NS = 16
NW = NC * NS
G = 16


def _sc_scores(node_emb, ctx_emb, vi, vj, neg_flat, K):
    B = vi.shape[0]
    b_per_w = B // NW
    n_per_w = b_per_w * K
    CH = G * K
    n_chunks = b_per_w // G
    mesh = plsc.VectorSubcoreMesh(core_axis_name="c", subcore_axis_name="s")

    @functools.partial(
        pl.kernel,
        mesh=mesh,
        out_type=jax.ShapeDtypeStruct((K + 1, B), jnp.float32),
        scratch_types=[
            pltpu.VMEM((b_per_w,), jnp.int32),
            pltpu.VMEM((b_per_w,), jnp.int32),
            pltpu.VMEM((n_per_w,), jnp.int32),
            pltpu.VMEM((G, D), jnp.float32),
            pltpu.VMEM((G, D), jnp.float32),
            pltpu.VMEM((G, D), jnp.float32),
            pltpu.VMEM((G, D), jnp.float32),
            pltpu.VMEM((CH, D), jnp.float32),
            pltpu.VMEM((CH, D), jnp.float32),
            pltpu.VMEM((D, G), jnp.float32),
            pltpu.VMEM((K + 1, b_per_w), jnp.float32),
            pltpu.SemaphoreType.DMA,
            pltpu.SemaphoreType.DMA,
            pltpu.SemaphoreType.DMA,
        ],
        compiler_params=pltpu.CompilerParams(
            disable_bounds_checks=True, needs_layout_passes=False),
    )
    def k(node_hbm, ctx_hbm, vi_hbm, vj_hbm, neg_hbm, out_hbm,
          vi_idx, vj_idx, ng_idx, vib0, vib1, vjb0, vjb1, nb0, nb1, vjt,
          sc_v, sem, sn0, sn1):
        wid = lax.axis_index("s") * NC + lax.axis_index("c")
        base = wid * b_per_w
        lanes = lax.iota(jnp.int32, G)
        lanes_k = lanes * K

        pltpu.async_copy(vi_hbm.at[pl.ds(base, b_per_w)], vi_idx, sem)
        pltpu.async_copy(vj_hbm.at[pl.ds(base, b_per_w)], vj_idx, sem)
        pltpu.async_copy(neg_hbm.at[pl.ds(base * K, n_per_w)], ng_idx, sem)
        pltpu.make_async_copy(vi_hbm.at[pl.ds(0, b_per_w)], vi_idx, sem).wait()
        pltpu.make_async_copy(vj_hbm.at[pl.ds(0, b_per_w)], vj_idx, sem).wait()
        pltpu.make_async_copy(neg_hbm.at[pl.ds(0, n_per_w)], ng_idx,
                              sem).wait()

        def fire_chunk(c, vib, vjb, ngb, s):
            idxv = vi_idx[pl.ds(c * G, G)]
            idxw = vj_idx[pl.ds(c * G, G)]
            for l in range(G):
                pltpu.async_copy(node_hbm.at[pl.ds(idxv[l], 1), :],
                                 vib.at[pl.ds(l, 1), :], s)
                pltpu.async_copy(ctx_hbm.at[pl.ds(idxw[l], 1), :],
                                 vjb.at[pl.ds(l, 1), :], s)
            co = c * CH

            @pl.loop(0, CH, step=G)
            def _(rr):
                idxn = ng_idx[pl.ds(co + rr, G)]
                for l in range(G):
                    pltpu.async_copy(ctx_hbm.at[pl.ds(idxn[l], 1), :],
                                     ngb.at[pl.ds(rr + l, 1), :], s)

        def drain_chunk(ngb, s):
            @pl.loop(0, CH + 2 * G)
            def _(r):
                pltpu.make_async_copy(
                    ctx_hbm.at[pl.ds(0, 1), :],
                    ngb.at[pl.ds(0, 1), :], s).wait()

        def compute(c, vib, vjb, ngb):
            g = c * G
            pos = jnp.zeros((G,), jnp.float32)
            for d in range(D):
                cd = jnp.full((G,), d, jnp.int32)
                vjc = plsc.load_gather(vjb, [lanes, cd])
                vic = plsc.load_gather(vib, [lanes, cd])
                vjt[d, :] = vjc
                pos = pos + vic * vjc
            sc_v[0, pl.ds(g, G)] = pos

            @pl.loop(0, K)
            def _(kk):
                rows_k = lanes_k + kk
                acc = jnp.zeros((G,), jnp.float32)
                for d in range(D):
                    cd = jnp.full((G,), d, jnp.int32)
                    nc = plsc.load_gather(ngb, [rows_k, cd])
                    acc = acc + nc * vjt[d, :]
                sc_v[kk + 1, pl.ds(g, G)] = acc

        fire_chunk(0, vib0, vjb0, nb0, sn0)

        @pl.loop(0, n_chunks)
        def _(c):
            @pl.when(c % 2 == 0)
            def _():
                @pl.when(c + 1 < n_chunks)
                def _():
                    fire_chunk(c + 1, vib1, vjb1, nb1, sn1)
                drain_chunk(nb0, sn0)
                compute(c, vib0, vjb0, nb0)

            @pl.when(c % 2 == 1)
            def _():
                @pl.when(c + 1 < n_chunks)
                def _():
                    fire_chunk(c + 1, vib0, vjb0, nb0, sn0)
                drain_chunk(nb1, sn1)
                compute(c, vib1, vjb1, nb1)

        pltpu.sync_copy(sc_v, out_hbm.at[:, pl.ds(base, b_per_w)])

    return k(node_emb, ctx_emb, vi, vj, neg_flat)


def _tc_loss(scores, B, K):
    BLK = 2048
    grid = B // BLK

    def body(s_ref, out_ref):
        i = pl.program_id(0)
        s = s_ref[...]
        pos = jax.nn.sigmoid(s[0, :])
        negl = jax.nn.log_sigmoid(-s[1:, :])
        part = jnp.sum(negl) + jnp.sum(pos)

        @pl.when(i == 0)
        def _():
            out_ref[0, 0] = 0.0

        out_ref[0, 0] += part

    out = pl.pallas_call(
        body,
        grid=(grid,),
        in_specs=[pl.BlockSpec((K + 1, BLK), lambda i: (0, i))],
        out_specs=pl.BlockSpec(memory_space=pltpu.SMEM),
        out_shape=jax.ShapeDtypeStruct((1, 1), jnp.float32),
    )(scores)
    return out[0, 0]


@jax.jit
def kernel(v_i, v_j, negative_samples, node_embeddings, context_embeddings):
    B, K = negative_samples.shape
    neg_flat = negative_samples.reshape(-1)
    scores = _sc_scores(node_embeddings, context_embeddings, v_i, v_j,
                        neg_flat, K)
    total = _tc_loss(scores, B, K)
    return -(total / B)
